# baseline (speedup 1.0000x reference)
import functools

import jax
import jax.numpy as jnp
from jax import lax
from jax.experimental import pallas as pl
from jax.experimental.pallas import tpu as pltpu
from jax.experimental.pallas import tpu_sc as plsc

NC = 2
NS = 16
NW = NC * NS
CH = 128

N = 10000
D = 128
H = 64
C = 40
NP = 10240
NPH = NP // 2
E = 320000
CPT = 80
EP = NW * CPT * CH
RPT = NP // NS

_mesh = plsc.VectorSubcoreMesh(core_axis_name="c", subcore_axis_name="s")
_sc_params = pltpu.CompilerParams(use_tc_tiling_on_sc=False)


@functools.partial(
    pl.kernel,
    out_type=[jax.ShapeDtypeStruct((NP, 16), jnp.float32),
              jax.ShapeDtypeStruct((NP, 16), jnp.float32)],
    mesh=_mesh,
    scratch_types=[
        pltpu.VMEM((CPT, CH), jnp.int32),
        pltpu.VMEM((CH, 16), jnp.float32),
        pltpu.VMEM_SHARED((NP, 16), jnp.float32),
    ],
    compiler_params=_sc_params,
)
def _deg_kernel(dst_hbm, ones_hbm, zeros_hbm, d0_hbm, d1_hbm,
                idx_v, ones_v, acc_sh):
    c = lax.axis_index("c")
    s = lax.axis_index("s")
    w = s * NC + c
    pltpu.sync_copy(ones_hbm, ones_v)
    pltpu.sync_copy(dst_hbm.at[w], idx_v)
    pltpu.sync_copy(zeros_hbm.at[pl.ds(s * RPT, RPT)],
                    acc_sh.at[pl.ds(s * RPT, RPT)])
    plsc.subcore_barrier()

    @pl.loop(0, CPT)
    def _(i):
        pltpu.sync_copy(ones_v, acc_sh.at[idx_v.at[i]], add=True)

    plsc.subcore_barrier()

    @pl.when(c == 0)
    def _():
        pltpu.sync_copy(acc_sh.at[pl.ds(s * RPT, RPT)],
                        d0_hbm.at[pl.ds(s * RPT, RPT)])

    @pl.when(c == 1)
    def _():
        pltpu.sync_copy(acc_sh.at[pl.ds(s * RPT, RPT)],
                        d1_hbm.at[pl.ds(s * RPT, RPT)])


@functools.partial(
    pl.kernel,
    out_type=[jax.ShapeDtypeStruct((NP, H), jnp.float32),
              jax.ShapeDtypeStruct((NP, H), jnp.float32)],
    mesh=_mesh,
    scratch_types=[
        pltpu.VMEM((CPT, CH), jnp.int32),
        pltpu.VMEM((CPT, CH), jnp.int32),
        [pltpu.VMEM((CH, H), jnp.float32) for _ in range(8)],
        pltpu.VMEM_SHARED((NP, H), jnp.float32),
        [pltpu.SemaphoreType.DMA for _ in range(8)],
    ],
    compiler_params=_sc_params,
)
def _agg_kernel(g_hbm, src_hbm, dst_hbm, zeros_hbm, p0_hbm, p1_hbm,
                src_v, dst_v, rows, acc_sh, sems):
    c = lax.axis_index("c")
    s = lax.axis_index("s")
    w = s * NC + c
    pltpu.sync_copy(src_hbm.at[w], src_v)
    pltpu.sync_copy(dst_hbm.at[w], dst_v)
    pltpu.sync_copy(zeros_hbm.at[pl.ds(s * RPT, RPT)],
                    acc_sh.at[pl.ds(s * RPT, RPT)])
    plsc.subcore_barrier()

    for b in range(8):
        pltpu.async_copy(g_hbm.at[src_v.at[b]], rows[b], sems[b])

    @pl.loop(0, (CPT - 8) // 8)
    def _(j):
        i = 8 * j
        for b in range(8):
            pltpu.make_async_copy(g_hbm.at[src_v.at[i + b]],
                                  rows[b], sems[b]).wait()
            pltpu.sync_copy(rows[b], acc_sh.at[dst_v.at[i + b]], add=True)
            pltpu.async_copy(g_hbm.at[src_v.at[i + b + 8]], rows[b], sems[b])

    for b in range(8):
        i = CPT - 8 + b
        pltpu.make_async_copy(g_hbm.at[src_v.at[i]], rows[b], sems[b]).wait()
        pltpu.sync_copy(rows[b], acc_sh.at[dst_v.at[i]], add=True)

    plsc.subcore_barrier()

    @pl.when(c == 0)
    def _():
        pltpu.sync_copy(acc_sh.at[pl.ds(s * RPT, RPT)],
                        p0_hbm.at[pl.ds(s * RPT, RPT)])

    @pl.when(c == 1)
    def _():
        pltpu.sync_copy(acc_sh.at[pl.ds(s * RPT, RPT)],
                        p1_hbm.at[pl.ds(s * RPT, RPT)])


BNH = 640


def _mm1_body(x_ref, w_ref, dinv_ref, g_ref):
    g_ref[...] = jnp.dot(x_ref[...], w_ref[...],
                         preferred_element_type=jnp.float32) * dinv_ref[...]


def _mm1(x2, w1s, dinvp):
    return pl.pallas_call(
        _mm1_body,
        grid=(NPH // BNH,),
        in_specs=[
            pl.BlockSpec((BNH, 2 * D), lambda i: (i, 0)),
            pl.BlockSpec((2 * D, 2 * H), lambda i: (0, 0)),
            pl.BlockSpec((BNH, 2 * H), lambda i: (i, 0)),
        ],
        out_specs=pl.BlockSpec((BNH, 2 * H), lambda i: (i, 0)),
        out_shape=jax.ShapeDtypeStruct((NPH, 2 * H), jnp.float32),
    )(x2, w1s, dinvp)


def _mid_body(p0_ref, p1_ref, g_ref, dinv_ref, b_ref, w_ref, o_ref):
    dinv = dinv_ref[...]
    h = (p0_ref[...] + p1_ref[...] + g_ref[...]) * dinv + b_ref[...]
    h = jnp.maximum(h, 0.0)
    o_ref[...] = jnp.dot(h, w_ref[...],
                         preferred_element_type=jnp.float32) * dinv


def _mid(p0, p1, g, dinvp, bp, ws):
    return pl.pallas_call(
        _mid_body,
        grid=(NPH // BNH,),
        in_specs=[
            pl.BlockSpec((BNH, 2 * H), lambda i: (i, 0)),
            pl.BlockSpec((BNH, 2 * H), lambda i: (i, 0)),
            pl.BlockSpec((BNH, 2 * H), lambda i: (i, 0)),
            pl.BlockSpec((BNH, 2 * H), lambda i: (i, 0)),
            pl.BlockSpec((1, 2 * H), lambda i: (0, 0)),
            pl.BlockSpec((2 * H, 2 * H), lambda i: (0, 0)),
        ],
        out_specs=pl.BlockSpec((BNH, 2 * H), lambda i: (i, 0)),
        out_shape=jax.ShapeDtypeStruct((NPH, 2 * H), jnp.float32),
    )(p0, p1, g, dinvp, bp, ws)


def _logsoftmax40(o):
    col = lax.broadcasted_iota(jnp.int32, o.shape, 1)
    o = jnp.where(col < C, o, -1e30)
    m = jnp.max(o, axis=1, keepdims=True)
    e = jnp.where(col < C, jnp.exp(o - m), 0.0)
    return o - m - jnp.log(jnp.sum(e, axis=1, keepdims=True))


def _fin_body(p0_ref, p1_ref, g_ref, dinv_ref, b_ref, o_ref):
    o = (p0_ref[...] + p1_ref[...] + g_ref[...]) * dinv_ref[...] + b_ref[...]
    o_ref[...] = jnp.concatenate(
        [_logsoftmax40(o[:, :H]), _logsoftmax40(o[:, H:])], axis=1)


def _fin(p0, p1, g, dinvp, bp):
    return pl.pallas_call(
        _fin_body,
        grid=(NPH // BNH,),
        in_specs=[
            pl.BlockSpec((BNH, 2 * H), lambda i: (i, 0)),
            pl.BlockSpec((BNH, 2 * H), lambda i: (i, 0)),
            pl.BlockSpec((BNH, 2 * H), lambda i: (i, 0)),
            pl.BlockSpec((BNH, 2 * H), lambda i: (i, 0)),
            pl.BlockSpec((1, 2 * H), lambda i: (0, 0)),
        ],
        out_specs=pl.BlockSpec((BNH, 2 * H), lambda i: (i, 0)),
        out_shape=jax.ShapeDtypeStruct((NPH, 2 * H), jnp.float32),
    )(p0, p1, g, dinvp, bp)


def _blockdiag2(w):
    hi, ho = w.shape
    z = jnp.zeros((hi, ho), jnp.float32)
    return jnp.concatenate([
        jnp.concatenate([w, z], axis=1),
        jnp.concatenate([z, w], axis=1),
    ], axis=0)


def kernel(x, edge_index, W1, b1, W2, b2, W3, b3):
    pad = EP - E
    spread = (jnp.arange(pad, dtype=jnp.int32) % 128) + N
    src = jnp.concatenate([edge_index[0], spread]).reshape(NW, CPT, CH)
    dst = jnp.concatenate([edge_index[1], spread]).reshape(NW, CPT, CH)

    x2 = jnp.zeros((NP, D), jnp.float32).at[:N].set(x).reshape(NPH, 2 * D)
    zeros16 = jnp.zeros((NP, 16), jnp.float32)
    zeros64 = jnp.zeros((NP, H), jnp.float32)
    ones = jnp.ones((CH, 16), jnp.float32)

    W1s = _blockdiag2(W1)
    W2s = _blockdiag2(W2)
    W3s = _blockdiag2(jnp.zeros((H, H), jnp.float32).at[:, :C].set(W3))
    b1p = jnp.concatenate([b1, b1]).reshape(1, 2 * H)
    b2p = jnp.concatenate([b2, b2]).reshape(1, 2 * H)
    b3f = jnp.zeros((H,), jnp.float32).at[:C].set(b3)
    b3p = jnp.concatenate([b3f, b3f]).reshape(1, 2 * H)

    d0, d1 = _deg_kernel(dst, ones, zeros16)
    dinv = lax.rsqrt(d0[:, 0] + d1[:, 0] + 1.0)
    dinvp = jnp.broadcast_to(dinv.reshape(NPH, 2, 1),
                             (NPH, 2, H)).reshape(NPH, 2 * H)

    g = _mm1(x2, W1s, dinvp)
    p0, p1 = _agg_kernel(g.reshape(NP, H), src, dst, zeros64)
    g = _mid(p0.reshape(NPH, 2 * H), p1.reshape(NPH, 2 * H), g, dinvp,
             b1p, W2s)
    p0, p1 = _agg_kernel(g.reshape(NP, H), src, dst, zeros64)
    g = _mid(p0.reshape(NPH, 2 * H), p1.reshape(NPH, 2 * H), g, dinvp,
             b2p, W3s)
    p0, p1 = _agg_kernel(g.reshape(NP, H), src, dst, zeros64)
    out = _fin(p0.reshape(NPH, 2 * H), p1.reshape(NPH, 2 * H), g, dinvp, b3p)
    return out.reshape(NP, H)[:N, :C]


# speedup vs baseline: 1.0056x; 1.0056x over previous
import functools

import jax
import jax.numpy as jnp
from jax import lax
from jax.experimental import pallas as pl
from jax.experimental.pallas import tpu as pltpu
from jax.experimental.pallas import tpu_sc as plsc

NC = 2
NS = 16
NW = NC * NS
CH = 128

N = 10000
D = 128
H = 64
C = 40
NP = 10240
NPH = NP // 2
E = 320000
CPT = 80
EP = NW * CPT * CH
RPT = NP // NS

_mesh = plsc.VectorSubcoreMesh(core_axis_name="c", subcore_axis_name="s")
_sc_params = pltpu.CompilerParams(use_tc_tiling_on_sc=False)


@functools.partial(
    pl.kernel,
    out_type=[jax.ShapeDtypeStruct((NP, 16), jnp.float32),
              jax.ShapeDtypeStruct((NP, 16), jnp.float32)],
    mesh=_mesh,
    scratch_types=[
        pltpu.VMEM((CPT, CH), jnp.int32),
        pltpu.VMEM((CH, 16), jnp.float32),
        pltpu.VMEM_SHARED((NP, 16), jnp.float32),
    ],
    compiler_params=_sc_params,
)
def _deg_kernel(dst_hbm, ones_hbm, zeros_hbm, d0_hbm, d1_hbm,
                idx_v, ones_v, acc_sh):
    c = lax.axis_index("c")
    s = lax.axis_index("s")
    w = s * NC + c
    pltpu.sync_copy(ones_hbm, ones_v)
    pltpu.sync_copy(dst_hbm.at[w], idx_v)
    pltpu.sync_copy(zeros_hbm.at[pl.ds(s * RPT, RPT)],
                    acc_sh.at[pl.ds(s * RPT, RPT)])
    plsc.subcore_barrier()

    @pl.loop(0, CPT)
    def _(i):
        pltpu.sync_copy(ones_v, acc_sh.at[idx_v.at[i]], add=True)

    plsc.subcore_barrier()

    @pl.when(c == 0)
    def _():
        pltpu.sync_copy(acc_sh.at[pl.ds(s * RPT, RPT)],
                        d0_hbm.at[pl.ds(s * RPT, RPT)])

    @pl.when(c == 1)
    def _():
        pltpu.sync_copy(acc_sh.at[pl.ds(s * RPT, RPT)],
                        d1_hbm.at[pl.ds(s * RPT, RPT)])


@functools.partial(
    pl.kernel,
    out_type=[jax.ShapeDtypeStruct((NP, H), jnp.float32),
              jax.ShapeDtypeStruct((NP, H), jnp.float32)],
    mesh=_mesh,
    scratch_types=[
        pltpu.VMEM((CPT, CH), jnp.int32),
        pltpu.VMEM((CPT, CH), jnp.int32),
        [pltpu.VMEM((CH, H), jnp.float32) for _ in range(4)],
        pltpu.VMEM_SHARED((NP, H), jnp.float32),
        [pltpu.SemaphoreType.DMA for _ in range(4)],
    ],
    compiler_params=_sc_params,
)
def _agg_kernel(g_hbm, src_hbm, dst_hbm, zeros_hbm, p0_hbm, p1_hbm,
                src_v, dst_v, rows, acc_sh, sems):
    c = lax.axis_index("c")
    s = lax.axis_index("s")
    w = s * NC + c
    pltpu.sync_copy(src_hbm.at[w], src_v)
    pltpu.sync_copy(dst_hbm.at[w], dst_v)
    pltpu.sync_copy(zeros_hbm.at[pl.ds(s * RPT, RPT)],
                    acc_sh.at[pl.ds(s * RPT, RPT)])
    plsc.subcore_barrier()

    for b in range(4):
        pltpu.async_copy(g_hbm.at[src_v.at[b]], rows[b], sems[b])

    @pl.loop(0, (CPT - 4) // 4)
    def _(j):
        i = 4 * j
        for b in range(4):
            pltpu.make_async_copy(g_hbm.at[src_v.at[i + b]],
                                  rows[b], sems[b]).wait()
            pltpu.sync_copy(rows[b], acc_sh.at[dst_v.at[i + b]], add=True)
            pltpu.async_copy(g_hbm.at[src_v.at[i + b + 4]], rows[b], sems[b])

    for b in range(4):
        i = CPT - 4 + b
        pltpu.make_async_copy(g_hbm.at[src_v.at[i]], rows[b], sems[b]).wait()
        pltpu.sync_copy(rows[b], acc_sh.at[dst_v.at[i]], add=True)

    plsc.subcore_barrier()

    @pl.when(c == 0)
    def _():
        pltpu.sync_copy(acc_sh.at[pl.ds(s * RPT, RPT)],
                        p0_hbm.at[pl.ds(s * RPT, RPT)])

    @pl.when(c == 1)
    def _():
        pltpu.sync_copy(acc_sh.at[pl.ds(s * RPT, RPT)],
                        p1_hbm.at[pl.ds(s * RPT, RPT)])


BNH = 640


def _mm1_body(x_ref, w_ref, dinv_ref, g_ref):
    g_ref[...] = jnp.dot(x_ref[...], w_ref[...],
                         preferred_element_type=jnp.float32) * dinv_ref[...]


def _mm1(x2, w1s, dinvp):
    return pl.pallas_call(
        _mm1_body,
        grid=(NPH // BNH,),
        in_specs=[
            pl.BlockSpec((BNH, 2 * D), lambda i: (i, 0)),
            pl.BlockSpec((2 * D, 2 * H), lambda i: (0, 0)),
            pl.BlockSpec((BNH, 2 * H), lambda i: (i, 0)),
        ],
        out_specs=pl.BlockSpec((BNH, 2 * H), lambda i: (i, 0)),
        out_shape=jax.ShapeDtypeStruct((NPH, 2 * H), jnp.float32),
    )(x2, w1s, dinvp)


def _mid_body(p0_ref, p1_ref, g_ref, dinv_ref, b_ref, w_ref, o_ref):
    dinv = dinv_ref[...]
    h = (p0_ref[...] + p1_ref[...] + g_ref[...]) * dinv + b_ref[...]
    h = jnp.maximum(h, 0.0)
    o_ref[...] = jnp.dot(h, w_ref[...],
                         preferred_element_type=jnp.float32) * dinv


def _mid(p0, p1, g, dinvp, bp, ws):
    return pl.pallas_call(
        _mid_body,
        grid=(NPH // BNH,),
        in_specs=[
            pl.BlockSpec((BNH, 2 * H), lambda i: (i, 0)),
            pl.BlockSpec((BNH, 2 * H), lambda i: (i, 0)),
            pl.BlockSpec((BNH, 2 * H), lambda i: (i, 0)),
            pl.BlockSpec((BNH, 2 * H), lambda i: (i, 0)),
            pl.BlockSpec((1, 2 * H), lambda i: (0, 0)),
            pl.BlockSpec((2 * H, 2 * H), lambda i: (0, 0)),
        ],
        out_specs=pl.BlockSpec((BNH, 2 * H), lambda i: (i, 0)),
        out_shape=jax.ShapeDtypeStruct((NPH, 2 * H), jnp.float32),
    )(p0, p1, g, dinvp, bp, ws)


def _logsoftmax40(o):
    col = lax.broadcasted_iota(jnp.int32, o.shape, 1)
    o = jnp.where(col < C, o, -1e30)
    m = jnp.max(o, axis=1, keepdims=True)
    e = jnp.where(col < C, jnp.exp(o - m), 0.0)
    return o - m - jnp.log(jnp.sum(e, axis=1, keepdims=True))


def _fin_body(p0_ref, p1_ref, g_ref, dinv_ref, b_ref, o_ref):
    o = (p0_ref[...] + p1_ref[...] + g_ref[...]) * dinv_ref[...] + b_ref[...]
    o_ref[...] = jnp.concatenate(
        [_logsoftmax40(o[:, :H]), _logsoftmax40(o[:, H:])], axis=1)


def _fin(p0, p1, g, dinvp, bp):
    return pl.pallas_call(
        _fin_body,
        grid=(NPH // BNH,),
        in_specs=[
            pl.BlockSpec((BNH, 2 * H), lambda i: (i, 0)),
            pl.BlockSpec((BNH, 2 * H), lambda i: (i, 0)),
            pl.BlockSpec((BNH, 2 * H), lambda i: (i, 0)),
            pl.BlockSpec((BNH, 2 * H), lambda i: (i, 0)),
            pl.BlockSpec((1, 2 * H), lambda i: (0, 0)),
        ],
        out_specs=pl.BlockSpec((BNH, 2 * H), lambda i: (i, 0)),
        out_shape=jax.ShapeDtypeStruct((NPH, 2 * H), jnp.float32),
    )(p0, p1, g, dinvp, bp)


def _blockdiag2(w):
    hi, ho = w.shape
    z = jnp.zeros((hi, ho), jnp.float32)
    return jnp.concatenate([
        jnp.concatenate([w, z], axis=1),
        jnp.concatenate([z, w], axis=1),
    ], axis=0)


def kernel(x, edge_index, W1, b1, W2, b2, W3, b3):
    pad = EP - E
    spread = (jnp.arange(pad, dtype=jnp.int32) % 128) + N
    src = jnp.concatenate([edge_index[0], spread]).reshape(NW, CPT, CH)
    dst = jnp.concatenate([edge_index[1], spread]).reshape(NW, CPT, CH)

    x2 = jnp.zeros((NP, D), jnp.float32).at[:N].set(x).reshape(NPH, 2 * D)
    zeros16 = jnp.zeros((NP, 16), jnp.float32)
    zeros64 = jnp.zeros((NP, H), jnp.float32)
    ones = jnp.ones((CH, 16), jnp.float32)

    W1s = _blockdiag2(W1)
    W2s = _blockdiag2(W2)
    W3s = _blockdiag2(jnp.zeros((H, H), jnp.float32).at[:, :C].set(W3))
    b1p = jnp.concatenate([b1, b1]).reshape(1, 2 * H)
    b2p = jnp.concatenate([b2, b2]).reshape(1, 2 * H)
    b3f = jnp.zeros((H,), jnp.float32).at[:C].set(b3)
    b3p = jnp.concatenate([b3f, b3f]).reshape(1, 2 * H)

    d0, d1 = _deg_kernel(dst, ones, zeros16)
    dinv = lax.rsqrt(d0[:, 0] + d1[:, 0] + 1.0)
    dinvp = jnp.broadcast_to(dinv.reshape(NPH, 2, 1),
                             (NPH, 2, H)).reshape(NPH, 2 * H)

    g = _mm1(x2, W1s, dinvp)
    p0, p1 = _agg_kernel(g.reshape(NP, H), src, dst, zeros64)
    g = _mid(p0.reshape(NPH, 2 * H), p1.reshape(NPH, 2 * H), g, dinvp,
             b1p, W2s)
    p0, p1 = _agg_kernel(g.reshape(NP, H), src, dst, zeros64)
    g = _mid(p0.reshape(NPH, 2 * H), p1.reshape(NPH, 2 * H), g, dinvp,
             b2p, W3s)
    p0, p1 = _agg_kernel(g.reshape(NP, H), src, dst, zeros64)
    out = _fin(p0.reshape(NPH, 2 * H), p1.reshape(NPH, 2 * H), g, dinvp, b3p)
    return out.reshape(NP, H)[:N, :C]


# speedup vs baseline: 1.0635x; 1.0575x over previous
import functools

import jax
import jax.numpy as jnp
from jax import lax
from jax.experimental import pallas as pl
from jax.experimental.pallas import tpu as pltpu
from jax.experimental.pallas import tpu_sc as plsc

NC = 2
NS = 16
NW = NC * NS
CH = 128

N = 10000
D = 128
H = 64
C = 40
NP = 10240
NPH = NP // 2
E = 320000
CPT = 80
EP = NW * CPT * CH
RPT = NP // NS

_mesh = plsc.VectorSubcoreMesh(core_axis_name="c", subcore_axis_name="s")
_sc_params = pltpu.CompilerParams(use_tc_tiling_on_sc=False)


@functools.partial(
    pl.kernel,
    out_type=[jax.ShapeDtypeStruct((NP, 16), jnp.float32),
              jax.ShapeDtypeStruct((NP, 16), jnp.float32)],
    mesh=_mesh,
    scratch_types=[
        pltpu.VMEM((CPT, CH), jnp.int32),
        pltpu.VMEM((CH, 16), jnp.float32),
        pltpu.VMEM_SHARED((NP, 16), jnp.float32),
        pltpu.SemaphoreType.DMA,
    ],
    compiler_params=_sc_params,
)
def _deg_kernel(ei_hbm, ones_hbm, zeros_hbm, d0_hbm, d1_hbm,
                idx_v, ones_v, acc_sh, sem):
    c = lax.axis_index("c")
    s = lax.axis_index("s")
    w = s * NC + c
    pltpu.sync_copy(ones_hbm, ones_v)
    pltpu.sync_copy(ei_hbm.at[1, w], idx_v)
    pltpu.sync_copy(zeros_hbm.at[pl.ds(s * RPT, RPT)],
                    acc_sh.at[pl.ds(s * RPT, RPT)])
    plsc.subcore_barrier()

    @pl.loop(0, CPT)
    def _(i):
        pltpu.async_copy(ones_v, acc_sh.at[idx_v.at[i]], sem, add=True)

    @pl.loop(0, CPT)
    def _(i):
        pltpu.make_async_copy(ones_v, acc_sh.at[idx_v.at[0]], sem).wait()

    plsc.subcore_barrier()

    @pl.when(c == 0)
    def _():
        pltpu.sync_copy(acc_sh.at[pl.ds(s * RPT, RPT)],
                        d0_hbm.at[pl.ds(s * RPT, RPT)])

    @pl.when(c == 1)
    def _():
        pltpu.sync_copy(acc_sh.at[pl.ds(s * RPT, RPT)],
                        d1_hbm.at[pl.ds(s * RPT, RPT)])


@functools.partial(
    pl.kernel,
    out_type=[jax.ShapeDtypeStruct((NP, H), jnp.float32),
              jax.ShapeDtypeStruct((NP, H), jnp.float32)],
    mesh=_mesh,
    scratch_types=[
        pltpu.VMEM((CPT, CH), jnp.int32),
        pltpu.VMEM((CPT, CH), jnp.int32),
        [pltpu.VMEM((CH, H), jnp.float32) for _ in range(4)],
        pltpu.VMEM_SHARED((NP, H), jnp.float32),
        [pltpu.SemaphoreType.DMA for _ in range(4)],
    ],
    compiler_params=_sc_params,
)
def _agg_kernel(g_hbm, ei_hbm, zeros_hbm, p0_hbm, p1_hbm,
                src_v, dst_v, rows, acc_sh, sems):
    c = lax.axis_index("c")
    s = lax.axis_index("s")
    w = s * NC + c
    pltpu.sync_copy(ei_hbm.at[0, w], src_v)
    pltpu.sync_copy(ei_hbm.at[1, w], dst_v)
    pltpu.sync_copy(zeros_hbm.at[pl.ds(s * RPT, RPT)],
                    acc_sh.at[pl.ds(s * RPT, RPT)])
    plsc.subcore_barrier()

    for b in range(4):
        pltpu.async_copy(g_hbm.at[src_v.at[b]], rows[b], sems[b])

    @pl.loop(0, (CPT - 4) // 4)
    def _(j):
        i = 4 * j
        for b in range(4):
            pltpu.make_async_copy(g_hbm.at[src_v.at[i + b]],
                                  rows[b], sems[b]).wait()
            pltpu.sync_copy(rows[b], acc_sh.at[dst_v.at[i + b]], add=True)
            pltpu.async_copy(g_hbm.at[src_v.at[i + b + 4]], rows[b], sems[b])

    for b in range(4):
        i = CPT - 4 + b
        pltpu.make_async_copy(g_hbm.at[src_v.at[i]], rows[b], sems[b]).wait()
        pltpu.sync_copy(rows[b], acc_sh.at[dst_v.at[i]], add=True)

    plsc.subcore_barrier()

    @pl.when(c == 0)
    def _():
        pltpu.sync_copy(acc_sh.at[pl.ds(s * RPT, RPT)],
                        p0_hbm.at[pl.ds(s * RPT, RPT)])

    @pl.when(c == 1)
    def _():
        pltpu.sync_copy(acc_sh.at[pl.ds(s * RPT, RPT)],
                        p1_hbm.at[pl.ds(s * RPT, RPT)])


BNH = 640


def _mm1_body(x_ref, w_ref, dinv_ref, g_ref):
    g_ref[...] = jnp.dot(x_ref[...], w_ref[...],
                         preferred_element_type=jnp.float32) * dinv_ref[...]


def _mm1(x2, w1s, dinvp):
    return pl.pallas_call(
        _mm1_body,
        grid=(NPH // BNH,),
        in_specs=[
            pl.BlockSpec((BNH, 2 * D), lambda i: (i, 0)),
            pl.BlockSpec((2 * D, 2 * H), lambda i: (0, 0)),
            pl.BlockSpec((BNH, 2 * H), lambda i: (i, 0)),
        ],
        out_specs=pl.BlockSpec((BNH, 2 * H), lambda i: (i, 0)),
        out_shape=jax.ShapeDtypeStruct((NPH, 2 * H), jnp.float32),
    )(x2, w1s, dinvp)


def _mid_body(p0_ref, p1_ref, g_ref, dinv_ref, b_ref, w_ref, o_ref):
    dinv = dinv_ref[...]
    h = (p0_ref[...] + p1_ref[...] + g_ref[...]) * dinv + b_ref[...]
    h = jnp.maximum(h, 0.0)
    o_ref[...] = jnp.dot(h, w_ref[...],
                         preferred_element_type=jnp.float32) * dinv


def _mid(p0, p1, g, dinvp, bp, ws):
    return pl.pallas_call(
        _mid_body,
        grid=(NPH // BNH,),
        in_specs=[
            pl.BlockSpec((BNH, 2 * H), lambda i: (i, 0)),
            pl.BlockSpec((BNH, 2 * H), lambda i: (i, 0)),
            pl.BlockSpec((BNH, 2 * H), lambda i: (i, 0)),
            pl.BlockSpec((BNH, 2 * H), lambda i: (i, 0)),
            pl.BlockSpec((1, 2 * H), lambda i: (0, 0)),
            pl.BlockSpec((2 * H, 2 * H), lambda i: (0, 0)),
        ],
        out_specs=pl.BlockSpec((BNH, 2 * H), lambda i: (i, 0)),
        out_shape=jax.ShapeDtypeStruct((NPH, 2 * H), jnp.float32),
    )(p0, p1, g, dinvp, bp, ws)


def _logsoftmax40(o):
    col = lax.broadcasted_iota(jnp.int32, o.shape, 1)
    o = jnp.where(col < C, o, -1e30)
    m = jnp.max(o, axis=1, keepdims=True)
    e = jnp.where(col < C, jnp.exp(o - m), 0.0)
    return o - m - jnp.log(jnp.sum(e, axis=1, keepdims=True))


def _fin_body(p0_ref, p1_ref, g_ref, dinv_ref, b_ref, o_ref):
    o = (p0_ref[...] + p1_ref[...] + g_ref[...]) * dinv_ref[...] + b_ref[...]
    o_ref[...] = jnp.concatenate(
        [_logsoftmax40(o[:, :H]), _logsoftmax40(o[:, H:])], axis=1)


def _fin(p0, p1, g, dinvp, bp):
    return pl.pallas_call(
        _fin_body,
        grid=(NPH // BNH,),
        in_specs=[
            pl.BlockSpec((BNH, 2 * H), lambda i: (i, 0)),
            pl.BlockSpec((BNH, 2 * H), lambda i: (i, 0)),
            pl.BlockSpec((BNH, 2 * H), lambda i: (i, 0)),
            pl.BlockSpec((BNH, 2 * H), lambda i: (i, 0)),
            pl.BlockSpec((1, 2 * H), lambda i: (0, 0)),
        ],
        out_specs=pl.BlockSpec((BNH, 2 * H), lambda i: (i, 0)),
        out_shape=jax.ShapeDtypeStruct((NPH, 2 * H), jnp.float32),
    )(p0, p1, g, dinvp, bp)


def _blockdiag2(w):
    hi, ho = w.shape
    z = jnp.zeros((hi, ho), jnp.float32)
    return jnp.concatenate([
        jnp.concatenate([w, z], axis=1),
        jnp.concatenate([z, w], axis=1),
    ], axis=0)


def kernel(x, edge_index, W1, b1, W2, b2, W3, b3):
    pad = EP - E
    spread = (jnp.arange(pad, dtype=jnp.int32) % 128) + N
    ei = jnp.concatenate(
        [edge_index, jnp.broadcast_to(spread, (2, pad))],
        axis=1).reshape(2, NW, CPT, CH)

    x2 = jnp.zeros((NP, D), jnp.float32).at[:N].set(x).reshape(NPH, 2 * D)
    zeros16 = jnp.zeros((NP, 16), jnp.float32)
    zeros64 = jnp.zeros((NP, H), jnp.float32)
    ones = jnp.ones((CH, 16), jnp.float32)

    W1s = _blockdiag2(W1)
    W2s = _blockdiag2(W2)
    W3s = _blockdiag2(jnp.zeros((H, H), jnp.float32).at[:, :C].set(W3))
    b1p = jnp.concatenate([b1, b1]).reshape(1, 2 * H)
    b2p = jnp.concatenate([b2, b2]).reshape(1, 2 * H)
    b3f = jnp.zeros((H,), jnp.float32).at[:C].set(b3)
    b3p = jnp.concatenate([b3f, b3f]).reshape(1, 2 * H)

    d0, d1 = _deg_kernel(ei, ones, zeros16)
    deg8 = d0.reshape(NP // 8, 128)[:, ::16] + d1.reshape(NP // 8, 128)[:, ::16]
    dinv = lax.rsqrt(deg8.reshape(NP) + 1.0)
    dinvp = jnp.broadcast_to(dinv.reshape(NPH, 2, 1),
                             (NPH, 2, H)).reshape(NPH, 2 * H)

    g = _mm1(x2, W1s, dinvp)
    p0, p1 = _agg_kernel(g.reshape(NP, H), ei, zeros64)
    g = _mid(p0.reshape(NPH, 2 * H), p1.reshape(NPH, 2 * H), g, dinvp,
             b1p, W2s)
    p0, p1 = _agg_kernel(g.reshape(NP, H), ei, zeros64)
    g = _mid(p0.reshape(NPH, 2 * H), p1.reshape(NPH, 2 * H), g, dinvp,
             b2p, W3s)
    p0, p1 = _agg_kernel(g.reshape(NP, H), ei, zeros64)
    out = _fin(p0.reshape(NPH, 2 * H), p1.reshape(NPH, 2 * H), g, dinvp, b3p)
    return out.reshape(NP, H)[:N, :C]
